# baseline (speedup 1.0000x reference)
import functools

import jax
import jax.numpy as jnp
from jax import lax
from jax.experimental import pallas as pl
from jax.experimental.pallas import tpu as pltpu
from jax.experimental.pallas import tpu_sc as plsc

L = 16
NC, NS = 2, 16
NW = NC * NS
B, Q, C = 64, 2048, 256
ROWS = B * Q
RPW = ROWS // NW
CHUNK = 64
NCHUNK = RPW // CHUNK
GROUPS = CHUNK // L
JCH = C // L
BIG = 2 ** 30


def _row_maxidx(logv, row):
    vs = [logv[row, pl.ds(L * j, L)] for j in range(JCH)]

    def chain(j0, n):
        m = vs[j0]
        ji = jnp.full((L,), j0, jnp.int32)
        for j in range(j0 + 1, j0 + n):
            gt = vs[j] > m
            m = jnp.maximum(m, vs[j])
            ji = jnp.where(gt, jnp.full((L,), j, jnp.int32), ji)
        return m, ji

    def merge(x, y):
        (mx, jx), (my, jy) = x, y
        return jnp.maximum(mx, my), jnp.where(my > mx, jy, jx)

    c0, c1, c2, c3 = chain(0, 4), chain(4, 4), chain(8, 4), chain(12, 4)
    return merge(merge(c0, c1), merge(c2, c3))


def _group(logv, bxv, bxov, labv, mbuf, fbuf, flagv, cntv, g):
    rbase = g * L
    iota = lax.iota(jnp.int32, L)
    for r in range(L):
        m, ji = _row_maxidx(logv, rbase + r)
        fidx = (ji << 4) | iota
        mbuf[pl.ds(r * L, L)] = m
        fbuf[pl.ds(r * L, L)] = fidx

    tidx = iota << 4
    cols = []
    for k in range(L):
        cols.append(plsc.load_gather(mbuf, [tidx + k]))
    gm = cols[0]
    for k in range(1, L):
        gm = jnp.maximum(gm, cols[k])
    cand = jnp.full((L,), BIG, jnp.int32)
    for k in range(L):
        f = plsc.load_gather(fbuf, [tidx + k])
        cand = jnp.minimum(cand, jnp.where(cols[k] == gm, f, jnp.full((L,), BIG, jnp.int32)))

    valid = gm > 0.0
    labels16 = jnp.where(valid, cand, jnp.full((L,), -1, jnp.int32))
    labv[pl.ds(rbase, L)] = labels16
    flags = jnp.where(valid, jnp.full((L,), 1.0, jnp.float32), jnp.full((L,), 0.0, jnp.float32))
    cntv[...] = cntv[...] + flags
    flagv[...] = flags
    qrows = iota >> 2
    qcols = iota & 3
    for q in range(4):
        fl = plsc.load_gather(flagv, [4 * q + qrows])
        rows = rbase + 4 * q + qrows
        b = plsc.load_gather(bxv, [rows, qcols])
        plsc.store_scatter(bxov, [rows, qcols], b * fl)


def _body(logits_hbm, boxes_hbm, labels_hbm, boxesout_hbm, cnt_hbm,
          log0, log1, bx0, bx1, bxo0, bxo1, lab0, lab1,
          mbuf, fbuf, flagv, cntv, si0, si1, so0, so1):
    cid = lax.axis_index("c")
    sid = lax.axis_index("s")
    wid = sid * NC + cid
    row0 = wid * RPW
    cntv[...] = jnp.zeros((L,), jnp.float32)

    def start_in(ci, logb, bxb, sem):
        crow = row0 + ci * CHUNK
        for h in range(4):
            qc = CHUNK // 4
            pltpu.async_copy(
                logits_hbm.at[pl.ds(crow + h * qc, qc), :],
                logb.at[pl.ds(h * qc, qc), :], sem)
        pltpu.async_copy(boxes_hbm.at[pl.ds(crow, CHUNK), :], bxb, sem)

    def wait_in(logb, bxb, sem):
        for h in range(4):
            qc = CHUNK // 4
            pltpu.make_async_copy(
                logits_hbm.at[pl.ds(0, qc), :],
                logb.at[pl.ds(h * qc, qc), :], sem).wait()
        pltpu.make_async_copy(boxes_hbm.at[pl.ds(0, CHUNK), :], bxb, sem).wait()

    def start_out(ci, labb, bxob, sem):
        crow = row0 + ci * CHUNK
        pltpu.async_copy(labb, labels_hbm.at[pl.ds(crow, CHUNK)], sem)
        pltpu.async_copy(bxob, boxesout_hbm.at[pl.ds(crow, CHUNK), :], sem)

    def wait_out(labb, bxob, sem):
        pltpu.make_async_copy(labb, labels_hbm.at[pl.ds(0, CHUNK)], sem).wait()
        pltpu.make_async_copy(bxob, boxesout_hbm.at[pl.ds(0, CHUNK), :], sem).wait()

    def compute(logb, bxb, bxob, labb):
        def g_body(g, c2):
            _group(logb, bxb, bxob, labb, mbuf, fbuf, flagv, cntv, g)
            return c2

        lax.fori_loop(0, GROUPS, g_body, 0)

    start_in(0, log0, bx0, si0)

    def pair(p, carry):
        c0 = 2 * p
        start_in(c0 + 1, log1, bx1, si1)
        wait_in(log0, bx0, si0)

        @pl.when(p > 0)
        def _():
            wait_out(lab0, bxo0, so0)

        compute(log0, bx0, bxo0, lab0)
        start_out(c0, lab0, bxo0, so0)
        start_in(jnp.minimum(c0 + 2, NCHUNK - 1), log0, bx0, si0)
        wait_in(log1, bx1, si1)

        @pl.when(p > 0)
        def _():
            wait_out(lab1, bxo1, so1)

        compute(log1, bx1, bxo1, lab1)
        start_out(c0 + 1, lab1, bxo1, so1)
        return carry

    lax.fori_loop(0, NCHUNK // 2, pair, 0)
    wait_in(log0, bx0, si0)
    wait_out(lab0, bxo0, so0)
    wait_out(lab1, bxo1, so1)
    pltpu.sync_copy(cntv, cnt_hbm.at[wid])


_sc_call = functools.partial(
    pl.kernel,
    mesh=plsc.VectorSubcoreMesh(core_axis_name="c", subcore_axis_name="s"),
    compiler_params=pltpu.CompilerParams(needs_layout_passes=False),
    out_type=[
        jax.ShapeDtypeStruct((ROWS,), jnp.int32),
        jax.ShapeDtypeStruct((ROWS, 4), jnp.float32),
        jax.ShapeDtypeStruct((NW, L), jnp.float32),
    ],
    scratch_types=[
        pltpu.VMEM((CHUNK, C), jnp.float32),
        pltpu.VMEM((CHUNK, C), jnp.float32),
        pltpu.VMEM((CHUNK, 4), jnp.float32),
        pltpu.VMEM((CHUNK, 4), jnp.float32),
        pltpu.VMEM((CHUNK, 4), jnp.float32),
        pltpu.VMEM((CHUNK, 4), jnp.float32),
        pltpu.VMEM((CHUNK,), jnp.int32),
        pltpu.VMEM((CHUNK,), jnp.int32),
        pltpu.VMEM((L * L,), jnp.float32),
        pltpu.VMEM((L * L,), jnp.int32),
        pltpu.VMEM((L,), jnp.float32),
        pltpu.VMEM((L,), jnp.float32),
        pltpu.SemaphoreType.DMA,
        pltpu.SemaphoreType.DMA,
        pltpu.SemaphoreType.DMA,
        pltpu.SemaphoreType.DMA,
    ],
)(_body)


def kernel(pred_logits, pred_boxes):
    logits = pred_logits.reshape(ROWS, C)
    boxes = pred_boxes.reshape(ROWS, 4)
    labels_flat, boxes_flat, cnts = _sc_call(logits, boxes)
    labels = labels_flat.reshape(B, Q)
    boxes_out = boxes_flat.reshape(B, Q, 4)
    num_boxes = jnp.maximum(jnp.sum(cnts), 1.0)
    return labels, boxes_out, num_boxes
